# baseline (speedup 1.0000x reference)
import functools

import jax
import jax.numpy as jnp
from jax import lax
from jax.experimental import pallas as pl
from jax.experimental.pallas import tpu as pltpu
from jax.experimental.pallas import tpu_sc as plsc

MASKED = -1000000000000.0

_NC = 2
_NS = 16
_NW = _NC * _NS
_LANES = 16


def _sc_mask_builder(v_pad, n_pad, chunk):
    mesh = plsc.VectorSubcoreMesh(core_axis_name="c", subcore_axis_name="s")

    @functools.partial(
        pl.kernel,
        mesh=mesh,
        out_type=(
            jax.ShapeDtypeStruct((v_pad,), jnp.float32),
            jax.ShapeDtypeStruct((v_pad,), jnp.float32),
        ),
        scratch_types=[
            pltpu.VMEM((n_pad,), jnp.int32),
            pltpu.VMEM((chunk,), jnp.float32),
        ],
        compiler_params=pltpu.CompilerParams(needs_layout_passes=False),
    )
    def sc_mask(vids_hbm, nids_hbm, vmask_hbm, nmask_hbm, ids_v, chunk_v):
        c = lax.axis_index("c")
        s = lax.axis_index("s")
        wid = s * _NC + c
        base = pl.multiple_of(wid * chunk, 8)

        zeros16 = jnp.zeros((_LANES,), jnp.float32)
        ones16 = jnp.ones((_LANES,), jnp.float32)

        for ids_hbm, mask_hbm in ((vids_hbm, vmask_hbm), (nids_hbm, nmask_hbm)):
            pltpu.sync_copy(ids_hbm, ids_v)

            def zero_body(i, _):
                chunk_v[pl.ds(i * _LANES, _LANES)] = zeros16
                return 0

            lax.fori_loop(0, chunk // _LANES, zero_body, 0)

            def scatter_body(j, _):
                ids16 = ids_v[pl.ds(j * _LANES, _LANES)]
                local = ids16 - base
                in_range = (local >= 0) & (local < chunk)
                safe = jnp.where(in_range, local, 0)
                plsc.store_scatter(chunk_v, [safe], ones16, mask=in_range)
                return 0

            lax.fori_loop(0, n_pad // _LANES, scatter_body, 0)

            pltpu.sync_copy(chunk_v, mask_hbm.at[pl.ds(base, chunk)])

    return sc_mask


def _tc_select_body(vmask_ref, nmask_ref, vlog_ref, nlog_ref, vout_ref, nout_ref):
    vout_ref[...] = jnp.where(vmask_ref[...] != 0.0, vlog_ref[...], MASKED)
    nout_ref[...] = jnp.where(nmask_ref[...] != 0.0, nlog_ref[...], MASKED)


def kernel(verb_logits, noun_logits, seen_verb_ids, seen_noun_ids):
    b, v = verb_logits.shape
    n = seen_verb_ids.shape[0]

    n_pad = ((n + _LANES - 1) // _LANES) * _LANES
    if n_pad != n:
        pad_v = jnp.broadcast_to(seen_verb_ids[:1], (n_pad - n,))
        pad_n = jnp.broadcast_to(seen_noun_ids[:1], (n_pad - n,))
        vids = jnp.concatenate([seen_verb_ids, pad_v])
        nids = jnp.concatenate([seen_noun_ids, pad_n])
    else:
        vids, nids = seen_verb_ids, seen_noun_ids

    blk = 2048
    grid = (v + blk - 1) // blk
    v_ceil = max(grid * blk, v)
    v_pad = ((v_ceil + _NW * 8 - 1) // (_NW * 8)) * (_NW * 8)
    chunk = v_pad // _NW

    vmask, nmask = _sc_mask_builder(v_pad, n_pad, chunk)(vids, nids)

    vmask2d = vmask.reshape(1, v_pad)
    nmask2d = nmask.reshape(1, v_pad)

    out = pl.pallas_call(
        _tc_select_body,
        grid=(grid,),
        in_specs=[
            pl.BlockSpec((1, blk), lambda i: (0, i)),
            pl.BlockSpec((1, blk), lambda i: (0, i)),
            pl.BlockSpec((b, blk), lambda i: (0, i)),
            pl.BlockSpec((b, blk), lambda i: (0, i)),
        ],
        out_specs=[
            pl.BlockSpec((b, blk), lambda i: (0, i)),
            pl.BlockSpec((b, blk), lambda i: (0, i)),
        ],
        out_shape=[
            jax.ShapeDtypeStruct((b, v), jnp.float32),
            jax.ShapeDtypeStruct((b, v), jnp.float32),
        ],
    )(vmask2d, nmask2d, verb_logits, noun_logits)

    return (out[0], out[1])


# speedup vs baseline: 1.0222x; 1.0222x over previous
import functools

import jax
import jax.numpy as jnp
from jax import lax
from jax.experimental import pallas as pl
from jax.experimental.pallas import tpu as pltpu
from jax.experimental.pallas import tpu_sc as plsc

MASKED = -1000000000000.0

_NC = 2
_NS = 16
_NW = _NC * _NS
_LANES = 16


def _sc_mask_builder(v, n, n_workers, chunk):
    mesh = plsc.VectorSubcoreMesh(core_axis_name="c", subcore_axis_name="s")

    @functools.partial(
        pl.kernel,
        mesh=mesh,
        out_type=(
            jax.ShapeDtypeStruct((v,), jnp.float32),
            jax.ShapeDtypeStruct((v,), jnp.float32),
        ),
        scratch_types=[
            pltpu.VMEM((n,), jnp.int32),
            pltpu.VMEM((chunk,), jnp.float32),
        ],
        compiler_params=pltpu.CompilerParams(needs_layout_passes=False),
    )
    def sc_mask(vids_hbm, nids_hbm, vmask_hbm, nmask_hbm, ids_v, chunk_v):
        c = lax.axis_index("c")
        s = lax.axis_index("s")
        wid = s * _NC + c

        @pl.when(wid < n_workers)
        def _():
            base = pl.multiple_of(wid * chunk, 8)
            zeros16 = jnp.zeros((_LANES,), jnp.float32)
            ones16 = jnp.ones((_LANES,), jnp.float32)

            n_full = n // _LANES
            tail = n % _LANES

            for ids_hbm, mask_hbm in ((vids_hbm, vmask_hbm), (nids_hbm, nmask_hbm)):
                pltpu.sync_copy(ids_hbm, ids_v)

                def zero_body(i, _):
                    chunk_v[pl.ds(i * _LANES, _LANES)] = zeros16
                    return 0

                lax.fori_loop(0, chunk // _LANES, zero_body, 0)

                def scatter_at(off):
                    ids16 = ids_v[pl.ds(off, _LANES)]
                    local = ids16 - base
                    in_range = (local >= 0) & (local < chunk)
                    safe = jnp.where(in_range, local, 0)
                    plsc.store_scatter(chunk_v, [safe], ones16, mask=in_range)

                def scatter_body(j, _):
                    scatter_at(j * _LANES)
                    return 0

                lax.fori_loop(0, n_full, scatter_body, 0)
                if tail:
                    scatter_at(n - _LANES)

                pltpu.sync_copy(chunk_v, mask_hbm.at[pl.ds(base, chunk)])

    return sc_mask


def _tc_select_body(vmask_ref, nmask_ref, vlog_ref, nlog_ref, vout_ref, nout_ref):
    vout_ref[...] = jnp.where(vmask_ref[...] != 0.0, vlog_ref[...], MASKED)
    nout_ref[...] = jnp.where(nmask_ref[...] != 0.0, nlog_ref[...], MASKED)


def kernel(verb_logits, noun_logits, seen_verb_ids, seen_noun_ids):
    b, v = verb_logits.shape
    n = seen_verb_ids.shape[0]

    n_workers = 1
    for w in range(_NW, 0, -1):
        if v % w == 0 and (v // w) % 8 == 0:
            n_workers = w
            break
    chunk = v // n_workers

    vmask, nmask = _sc_mask_builder(v, n, n_workers, chunk)(
        seen_verb_ids, seen_noun_ids
    )

    vmask2d = vmask.reshape(1, v)
    nmask2d = nmask.reshape(1, v)

    rows = 8
    grid = b // rows

    out = pl.pallas_call(
        _tc_select_body,
        grid=(grid,),
        in_specs=[
            pl.BlockSpec((1, v), lambda i: (0, 0)),
            pl.BlockSpec((1, v), lambda i: (0, 0)),
            pl.BlockSpec((rows, v), lambda i: (i, 0)),
            pl.BlockSpec((rows, v), lambda i: (i, 0)),
        ],
        out_specs=[
            pl.BlockSpec((rows, v), lambda i: (i, 0)),
            pl.BlockSpec((rows, v), lambda i: (i, 0)),
        ],
        out_shape=[
            jax.ShapeDtypeStruct((b, v), jnp.float32),
            jax.ShapeDtypeStruct((b, v), jnp.float32),
        ],
        compiler_params=pltpu.CompilerParams(
            dimension_semantics=("arbitrary",),
        ),
    )(vmask2d, nmask2d, verb_logits, noun_logits)

    return (out[0], out[1])


# speedup vs baseline: 2.1847x; 2.1373x over previous
import functools

import jax
import jax.numpy as jnp
from jax import lax
from jax.experimental import pallas as pl
from jax.experimental.pallas import tpu as pltpu
from jax.experimental.pallas import tpu_sc as plsc

MASKED = -1000000000000.0

_NC = 2
_NS = 16
_NW = _NC * _NS
_LANES = 16


def _sc_mask_builder(v, n, n_workers, chunk):
    mesh = plsc.VectorSubcoreMesh(core_axis_name="c", subcore_axis_name="s")

    @functools.partial(
        pl.kernel,
        mesh=mesh,
        out_type=(
            jax.ShapeDtypeStruct((v,), jnp.float32),
            jax.ShapeDtypeStruct((v,), jnp.float32),
        ),
        scratch_types=[
            pltpu.VMEM((n,), jnp.int32),
            pltpu.VMEM((chunk,), jnp.float32),
        ],
        compiler_params=pltpu.CompilerParams(needs_layout_passes=False),
    )
    def sc_mask(vids_hbm, nids_hbm, vmask_hbm, nmask_hbm, ids_v, chunk_v):
        c = lax.axis_index("c")
        s = lax.axis_index("s")
        wid = s * _NC + c

        @pl.when(wid < n_workers)
        def _():
            base = pl.multiple_of(wid * chunk, 8)
            zeros16 = jnp.zeros((_LANES,), jnp.float32)
            ones16 = jnp.ones((_LANES,), jnp.float32)

            n_full = n // _LANES
            tail = n % _LANES

            for ids_hbm, mask_hbm in ((vids_hbm, vmask_hbm), (nids_hbm, nmask_hbm)):
                pltpu.sync_copy(ids_hbm, ids_v)

                def zero_body(i, _):
                    chunk_v[pl.ds(i * _LANES, _LANES)] = zeros16
                    return 0

                lax.fori_loop(0, chunk // _LANES, zero_body, 0)

                def scatter_at(off):
                    ids16 = ids_v[pl.ds(off, _LANES)]
                    local = ids16 - base
                    in_range = (local >= 0) & (local < chunk)
                    safe = jnp.where(in_range, local, 0)
                    plsc.store_scatter(chunk_v, [safe], ones16, mask=in_range)

                def scatter_body(j, _):
                    scatter_at(j * _LANES)
                    return 0

                lax.fori_loop(0, n_full, scatter_body, 0)
                if tail:
                    scatter_at(n - _LANES)

                pltpu.sync_copy(chunk_v, mask_hbm.at[pl.ds(base, chunk)])

    return sc_mask


def _tc_select_body(vmask_ref, nmask_ref, vlog_ref, nlog_ref, vout_ref, nout_ref):
    vkeep = vmask_ref[0].T != 0.0
    nkeep = nmask_ref[0].T != 0.0
    vout_ref[...] = jnp.where(vkeep, vlog_ref[...], MASKED)
    nout_ref[...] = jnp.where(nkeep, nlog_ref[...], MASKED)


def kernel(verb_logits, noun_logits, seen_verb_ids, seen_noun_ids):
    b, v = verb_logits.shape
    n = seen_verb_ids.shape[0]

    n_workers = 1
    for w in range(_NW, 0, -1):
        if v % w == 0 and (v // w) % 8 == 0:
            n_workers = w
            break
    chunk = v // n_workers

    vmask, nmask = _sc_mask_builder(v, n, n_workers, chunk)(
        seen_verb_ids, seen_noun_ids
    )

    vlog_t = verb_logits.T
    nlog_t = noun_logits.T

    rows = 2000
    grid = v // rows

    vmask2d = vmask.reshape(grid, 1, rows)
    nmask2d = nmask.reshape(grid, 1, rows)

    out = pl.pallas_call(
        _tc_select_body,
        grid=(grid,),
        in_specs=[
            pl.BlockSpec((1, 1, rows), lambda i: (i, 0, 0)),
            pl.BlockSpec((1, 1, rows), lambda i: (i, 0, 0)),
            pl.BlockSpec((rows, b), lambda i: (i, 0)),
            pl.BlockSpec((rows, b), lambda i: (i, 0)),
        ],
        out_specs=[
            pl.BlockSpec((rows, b), lambda i: (i, 0)),
            pl.BlockSpec((rows, b), lambda i: (i, 0)),
        ],
        out_shape=[
            jax.ShapeDtypeStruct((v, b), jnp.float32),
            jax.ShapeDtypeStruct((v, b), jnp.float32),
        ],
        compiler_params=pltpu.CompilerParams(
            dimension_semantics=("arbitrary",),
        ),
    )(vmask2d, nmask2d, vlog_t, nlog_t)

    return (out[0].T, out[1].T)


# speedup vs baseline: 2.3672x; 1.0835x over previous
import functools

import jax
import jax.numpy as jnp
from jax import lax
from jax.experimental import pallas as pl
from jax.experimental.pallas import tpu as pltpu
from jax.experimental.pallas import tpu_sc as plsc

MASKED = -1000000000000.0

_NC = 2
_NS = 16
_NW = _NC * _NS
_LANES = 16


def _sc_mask_builder(v, n, n_workers, chunk):
    mesh = plsc.VectorSubcoreMesh(core_axis_name="c", subcore_axis_name="s")

    @functools.partial(
        pl.kernel,
        mesh=mesh,
        out_type=(
            jax.ShapeDtypeStruct((v,), jnp.float32),
            jax.ShapeDtypeStruct((v,), jnp.float32),
        ),
        scratch_types=[
            pltpu.VMEM((n,), jnp.int32),
            pltpu.VMEM((chunk,), jnp.float32),
        ],
        compiler_params=pltpu.CompilerParams(needs_layout_passes=False),
    )
    def sc_mask(vids_hbm, nids_hbm, vmask_hbm, nmask_hbm, ids_v, chunk_v):
        c = lax.axis_index("c")
        s = lax.axis_index("s")
        wid = s * _NC + c

        @pl.when(wid < n_workers)
        def _():
            base = pl.multiple_of(wid * chunk, 8)
            zeros16 = jnp.zeros((_LANES,), jnp.float32)
            ones16 = jnp.ones((_LANES,), jnp.float32)

            n_full = n // _LANES
            tail = n % _LANES

            for ids_hbm, mask_hbm in ((vids_hbm, vmask_hbm), (nids_hbm, nmask_hbm)):
                pltpu.sync_copy(ids_hbm, ids_v)

                def zero_body(i, _):
                    chunk_v[pl.ds(i * _LANES, _LANES)] = zeros16
                    return 0

                lax.fori_loop(0, chunk // _LANES, zero_body, 0)

                def scatter_at(off):
                    ids16 = ids_v[pl.ds(off, _LANES)]
                    local = ids16 - base
                    in_range = (local >= 0) & (local < chunk)
                    safe = jnp.where(in_range, local, 0)
                    plsc.store_scatter(chunk_v, [safe], ones16, mask=in_range)

                def scatter_body(j, _):
                    scatter_at(j * _LANES)
                    return 0

                lax.fori_loop(0, n_full, scatter_body, 0)
                if tail:
                    scatter_at(n - _LANES)

                pltpu.sync_copy(chunk_v, mask_hbm.at[pl.ds(base, chunk)])

    return sc_mask


def _tc_select_body(vmask_ref, nmask_ref, vlog_ref, nlog_ref, vout_ref, nout_ref):
    b = vlog_ref.shape[1]
    ones_row = jnp.ones((1, b), jnp.float32)
    dn = (((0,), (0,)), ((), ()))
    vb = jax.lax.dot_general(vmask_ref[0], ones_row, dn,
                             preferred_element_type=jnp.float32)
    nb = jax.lax.dot_general(nmask_ref[0], ones_row, dn,
                             preferred_element_type=jnp.float32)
    vout_ref[...] = jnp.where(vb != 0.0, vlog_ref[...], MASKED)
    nout_ref[...] = jnp.where(nb != 0.0, nlog_ref[...], MASKED)


def kernel(verb_logits, noun_logits, seen_verb_ids, seen_noun_ids):
    b, v = verb_logits.shape
    n = seen_verb_ids.shape[0]

    n_workers = 1
    for w in range(_NW, 0, -1):
        if v % w == 0 and (v // w) % 8 == 0:
            n_workers = w
            break
    chunk = v // n_workers

    vmask, nmask = _sc_mask_builder(v, n, n_workers, chunk)(
        seen_verb_ids, seen_noun_ids
    )

    vlog_t = verb_logits.T
    nlog_t = noun_logits.T

    rows = 2000
    grid = v // rows

    vmask2d = vmask.reshape(grid, 1, rows)
    nmask2d = nmask.reshape(grid, 1, rows)

    out = pl.pallas_call(
        _tc_select_body,
        grid=(grid,),
        in_specs=[
            pl.BlockSpec((1, 1, rows), lambda i: (i, 0, 0)),
            pl.BlockSpec((1, 1, rows), lambda i: (i, 0, 0)),
            pl.BlockSpec((rows, b), lambda i: (i, 0)),
            pl.BlockSpec((rows, b), lambda i: (i, 0)),
        ],
        out_specs=[
            pl.BlockSpec((rows, b), lambda i: (i, 0)),
            pl.BlockSpec((rows, b), lambda i: (i, 0)),
        ],
        out_shape=[
            jax.ShapeDtypeStruct((v, b), jnp.float32),
            jax.ShapeDtypeStruct((v, b), jnp.float32),
        ],
        compiler_params=pltpu.CompilerParams(
            dimension_semantics=("arbitrary",),
        ),
    )(vmask2d, nmask2d, vlog_t, nlog_t)

    return (out[0].T, out[1].T)
